# Initial kernel scaffold; baseline (speedup 1.0000x reference)
import functools

import jax
import jax.numpy as jnp
from jax import lax
from jax.experimental import pallas as pl
from jax.experimental.pallas import tpu as pltpu
from jax.experimental.pallas import tpu_sc as plsc

NB = 100
SCALE = 3.0
HALF = jnp.float32(SCALE / 2.0)
STEP = jnp.float32(SCALE / NB)
NVOX = NB * NB * NB
PAD_ROWS = 8
DUMMY = NVOX

NC, NS, L = 2, 16, 16
NW = NC * NS

C = 4096
GB = 128
NR = C // GB


def _make_sc_kernel(n_points):
    npw = n_points // NW
    nch = npw // C
    assert npw * NW == n_points and nch * C == npw

    mesh = plsc.VectorSubcoreMesh(core_axis_name="c", subcore_axis_name="s")

    @functools.partial(
        pl.kernel,
        out_type=jax.ShapeDtypeStruct((n_points, 4), jnp.float32),
        mesh=mesh,
        scratch_types=[
            pltpu.VMEM((C, 3), jnp.float32),
            pltpu.VMEM((NR, GB), jnp.int32),
            pltpu.VMEM((C, 4), jnp.float32),
            pltpu.SemaphoreType.DMA,
        ],
    )
    def sc_kernel(xyz_hbm, vox_hbm, out_hbm, xyz_v, idx_v, rows_v, sem):
        wid = lax.axis_index("s") * NC + lax.axis_index("c")

        iota = jax.lax.iota(jnp.int32, L)
        col0 = jnp.zeros((L,), jnp.int32)
        col1 = col0 + 1
        col2 = col0 + 2
        q4 = iota >> 2
        r4 = iota & 3
        is_d = r4 == 3

        @pl.loop(0, nch)
        def _chunk(ch):
            start = wid * npw + ch * C
            pltpu.sync_copy(xyz_hbm.at[pl.ds(start, C)], xyz_v)

            @pl.loop(0, NR)
            def _idx(r):
                for g in range(GB // L):
                    pt = iota + (r * GB + g * L)
                    x = plsc.load_gather(xyz_v, [pt, col0])
                    y = plsc.load_gather(xyz_v, [pt, col1])
                    z = plsc.load_gather(xyz_v, [pt, col2])
                    cond = ((jnp.abs(x) < HALF) & (jnp.abs(y) < HALF)
                            & (jnp.abs(z) < HALF))
                    ix = jnp.clip((x / STEP + 50.0).astype(jnp.int32), 0, NB - 1)
                    iy = jnp.clip((y / STEP + 50.0).astype(jnp.int32), 0, NB - 1)
                    iz = jnp.clip((z / STEP + 50.0).astype(jnp.int32), 0, NB - 1)
                    flat = (ix * NB + iy) * NB + iz
                    flat = jnp.where(cond, flat, DUMMY)
                    idx_v[r, pl.ds(g * L, L)] = flat

            copies = [
                pltpu.async_copy(
                    vox_hbm.at[idx_v.at[r]],
                    rows_v.at[pl.ds(r * GB, GB)],
                    sem,
                )
                for r in range(NR)
            ]
            for cp in copies:
                cp.wait()

            @pl.loop(0, C * 4 // L)
            def _act(k):
                pt = q4 + k * (L // 4)
                v = plsc.load_gather(rows_v, [pt, r4])
                sig = 1.0 / (1.0 + jnp.exp(-v))
                out = jnp.where(is_d, jnp.maximum(v, 0.0), sig)
                plsc.store_scatter(rows_v, [pt, r4], out)

            pltpu.sync_copy(rows_v, out_hbm.at[pl.ds(start, C)])

    return sc_kernel


_SC_KERNEL = _make_sc_kernel(4194304)


def kernel(xyz, voxels):
    vox = voxels.reshape(-1, 4)
    vox = jnp.concatenate([vox, jnp.zeros((PAD_ROWS, 4), jnp.float32)], axis=0)
    return _SC_KERNEL(xyz, vox)


# speedup vs baseline: 4.2452x; 4.2452x over previous
import dataclasses
import functools

import jax
import jax.numpy as jnp
import numpy as np
from jax import lax
from jax.experimental import pallas as pl
from jax.experimental.pallas import tpu as pltpu
from jax.experimental.pallas import tpu_sc as plsc

NB = 100
SCALE = 3.0
HALF = np.float32(SCALE / 2.0)
STEP = np.float32(SCALE / NB)
NVOX = NB * NB * NB
PAD_ROWS = 8
DUMMY = NVOX
ROW = 16

NC, NS, L = 2, 16, 16
NW = NC * NS

C = 2048
GB = 128
NR = C // GB


def _make_sc_kernel(n_points):
    npw = n_points // NW
    nch = npw // C
    assert npw * NW == n_points and nch * C == npw

    mesh = plsc.VectorSubcoreMesh(core_axis_name="c", subcore_axis_name="s")

    cp = pltpu.CompilerParams()
    fields = pltpu.CompilerParams.__dataclass_fields__
    if "needs_layout_passes" in fields:
        cp = dataclasses.replace(cp, needs_layout_passes=False)
    if "use_tc_tiling_on_sc" in fields:
        cp = dataclasses.replace(cp, use_tc_tiling_on_sc=False)

    @functools.partial(
        pl.kernel,
        out_type=jax.ShapeDtypeStruct((n_points, 4), jnp.float32),
        mesh=mesh,
        compiler_params=cp,
        scratch_types=[
            pltpu.VMEM((C, 3), jnp.float32),
            pltpu.VMEM((NR, GB), jnp.int32),
            pltpu.VMEM((C, ROW), jnp.float32),
            pltpu.VMEM((C, 4), jnp.float32),
            pltpu.SemaphoreType.DMA,
        ],
    )
    def sc_kernel(xyz_hbm, vox_hbm, out_hbm, xyz_v, idx_v, rows_v, out_v, sem):
        wid = lax.axis_index("s") * NC + lax.axis_index("c")

        iota = jax.lax.iota(jnp.int32, L)
        col0 = jnp.zeros((L,), jnp.int32)
        col1 = col0 + 1
        col2 = col0 + 2
        q4 = iota >> 2
        r4 = iota & 3
        is_d = r4 == 3

        @pl.loop(0, nch)
        def _chunk(ch):
            start = wid * npw + ch * C
            pltpu.sync_copy(xyz_hbm.at[pl.ds(start, C)], xyz_v)

            @pl.loop(0, NR)
            def _idx(r):
                for g in range(GB // L):
                    pt = iota + (r * GB + g * L)
                    x = plsc.load_gather(xyz_v, [pt, col0])
                    y = plsc.load_gather(xyz_v, [pt, col1])
                    z = plsc.load_gather(xyz_v, [pt, col2])
                    cond = ((jnp.abs(x) < HALF) & (jnp.abs(y) < HALF)
                            & (jnp.abs(z) < HALF))
                    ix = jnp.clip((x / STEP + 50.0).astype(jnp.int32), 0, NB - 1)
                    iy = jnp.clip((y / STEP + 50.0).astype(jnp.int32), 0, NB - 1)
                    iz = jnp.clip((z / STEP + 50.0).astype(jnp.int32), 0, NB - 1)
                    flat = (ix * NB + iy) * NB + iz
                    flat = jnp.where(cond, flat, DUMMY)
                    idx_v[r, pl.ds(g * L, L)] = flat

            copies = [
                pltpu.async_copy(
                    vox_hbm.at[idx_v.at[r]],
                    rows_v.at[pl.ds(r * GB, GB)],
                    sem,
                )
                for r in range(NR)
            ]
            for cp in copies:
                cp.wait()

            @pl.loop(0, C * 4 // L)
            def _act(k):
                pt = q4 + k * (L // 4)
                v = plsc.load_gather(rows_v, [pt, r4])
                sig = 1.0 / (1.0 + jnp.exp(-v))
                out = jnp.where(is_d, jnp.maximum(v, 0.0), sig)
                plsc.store_scatter(out_v, [pt, r4], out)

            pltpu.sync_copy(out_v, out_hbm.at[pl.ds(start, C)])

    return sc_kernel


_SC_KERNEL = _make_sc_kernel(4194304)


def kernel(xyz, voxels):
    vox = voxels.reshape(-1, 4)
    vox = jnp.pad(vox, ((0, PAD_ROWS), (0, ROW - 4)))
    return _SC_KERNEL(xyz, vox)


# speedup vs baseline: 4.3331x; 1.0207x over previous
import dataclasses
import functools

import jax
import jax.numpy as jnp
import numpy as np
from jax import lax
from jax.experimental import pallas as pl
from jax.experimental.pallas import tpu as pltpu
from jax.experimental.pallas import tpu_sc as plsc

NB = 100
SCALE = 3.0
HALF = np.float32(SCALE / 2.0)
STEP = np.float32(SCALE / NB)
NVOX = NB * NB * NB
PAD_ROWS = 8
DUMMY = NVOX
ROW = 16

NC, NS, L = 2, 16, 16
NW = NC * NS

C = 2048


def _make_sc_kernel(n_points):
    npw = n_points // NW
    nch = npw // C
    assert npw * NW == n_points and nch * C == npw

    mesh = plsc.VectorSubcoreMesh(core_axis_name="c", subcore_axis_name="s")

    cp = pltpu.CompilerParams()
    fields = pltpu.CompilerParams.__dataclass_fields__
    if "needs_layout_passes" in fields:
        cp = dataclasses.replace(cp, needs_layout_passes=False)
    if "use_tc_tiling_on_sc" in fields:
        cp = dataclasses.replace(cp, use_tc_tiling_on_sc=False)

    @functools.partial(
        pl.kernel,
        out_type=jax.ShapeDtypeStruct((n_points * 4,), jnp.float32),
        mesh=mesh,
        compiler_params=cp,
        scratch_types=[
            pltpu.VMEM((C * 3,), jnp.float32),
            pltpu.VMEM((C,), jnp.int32),
            pltpu.VMEM((C, ROW), jnp.float32),
            pltpu.VMEM((C * 4,), jnp.float32),
            pltpu.SemaphoreType.DMA,
        ],
    )
    def sc_kernel(xyz_hbm, vox_hbm, out_hbm, xyz_v, idx_v, rows_v, out_v, sem):
        wid = lax.axis_index("s") * NC + lax.axis_index("c")

        iota = jax.lax.iota(jnp.int32, L)
        iota3 = iota * 3
        q4 = iota >> 2
        r4 = iota & 3
        is_d = r4 == 3

        @pl.loop(0, nch)
        def _chunk(ch):
            start = wid * npw + ch * C
            pltpu.sync_copy(xyz_hbm.at[pl.ds(start * 3, C * 3)], xyz_v)

            @pl.loop(0, C // L)
            def _idx(g):
                p3 = iota3 + g * (3 * L)
                x = plsc.load_gather(xyz_v, [p3])
                y = plsc.load_gather(xyz_v, [p3 + 1])
                z = plsc.load_gather(xyz_v, [p3 + 2])
                cond = ((jnp.abs(x) < HALF) & (jnp.abs(y) < HALF)
                        & (jnp.abs(z) < HALF))
                ix = jnp.clip((x / STEP + 50.0).astype(jnp.int32), 0, NB - 1)
                iy = jnp.clip((y / STEP + 50.0).astype(jnp.int32), 0, NB - 1)
                iz = jnp.clip((z / STEP + 50.0).astype(jnp.int32), 0, NB - 1)
                flat = (ix * NB + iy) * NB + iz
                flat = jnp.where(cond, flat, DUMMY)
                idx_v[pl.ds(g * L, L)] = flat

            pltpu.async_copy(vox_hbm.at[idx_v], rows_v, sem).wait()

            @pl.loop(0, C * 4 // L)
            def _act(k):
                pt = q4 + k * (L // 4)
                v = plsc.load_gather(rows_v, [pt, r4])
                sig = 1.0 / (1.0 + jnp.exp(-v))
                out = jnp.where(is_d, jnp.maximum(v, 0.0), sig)
                out_v[pl.ds(k * L, L)] = out

            pltpu.sync_copy(out_v, out_hbm.at[pl.ds(start * 4, C * 4)])

    return sc_kernel


_N_POINTS = 4194304
_SC_KERNEL = _make_sc_kernel(_N_POINTS)


def kernel(xyz, voxels):
    vox = voxels.reshape(-1, 4)
    vox = jnp.pad(vox, ((0, PAD_ROWS), (0, ROW - 4)))
    out = _SC_KERNEL(xyz.reshape(-1), vox)
    return out.reshape(_N_POINTS, 4)


# speedup vs baseline: 7.1477x; 1.6496x over previous
import dataclasses
import functools

import jax
import jax.numpy as jnp
import numpy as np
from jax import lax
from jax.experimental import pallas as pl
from jax.experimental.pallas import tpu as pltpu
from jax.experimental.pallas import tpu_sc as plsc

NB = 100
SCALE = 3.0
HALF = np.float32(SCALE / 2.0)
STEP = np.float32(SCALE / NB)
NVOX = NB * NB * NB
PAD_ROWS = 8
DUMMY = NVOX
ROW = 16

NC, NS, L = 2, 16, 16
NW = NC * NS

C = 2048


def _make_sc_kernel(n_points):
    npw = n_points // NW
    nch = npw // C
    assert npw * NW == n_points and nch * C == npw

    mesh = plsc.VectorSubcoreMesh(core_axis_name="c", subcore_axis_name="s")

    cp = pltpu.CompilerParams()
    fields = pltpu.CompilerParams.__dataclass_fields__
    if "needs_layout_passes" in fields:
        cp = dataclasses.replace(cp, needs_layout_passes=False)
    if "use_tc_tiling_on_sc" in fields:
        cp = dataclasses.replace(cp, use_tc_tiling_on_sc=False)

    @functools.partial(
        pl.kernel,
        out_type=jax.ShapeDtypeStruct((4, n_points), jnp.float32),
        mesh=mesh,
        compiler_params=cp,
        scratch_types=[
            pltpu.VMEM((3, C), jnp.float32),
            pltpu.VMEM((C,), jnp.int32),
            pltpu.VMEM((C, ROW), jnp.float32),
            pltpu.VMEM((4, C), jnp.float32),
            pltpu.SemaphoreType.DMA,
        ],
    )
    def sc_kernel(xyz_hbm, vox_hbm, out_hbm, xyz_v, idx_v, rows_v, out_v, sem):
        wid = lax.axis_index("s") * NC + lax.axis_index("c")

        iota = jax.lax.iota(jnp.int32, L)
        csplat = [jnp.full((L,), c, jnp.int32) for c in range(4)]

        @pl.loop(0, nch)
        def _chunk(ch):
            start = wid * npw + ch * C
            for a in range(3):
                pltpu.sync_copy(xyz_hbm.at[a, pl.ds(start, C)], xyz_v.at[a])

            @pl.loop(0, C // L)
            def _idx(g):
                s = pl.ds(g * L, L)
                x = xyz_v[0, s]
                y = xyz_v[1, s]
                z = xyz_v[2, s]
                cond = ((jnp.abs(x) < HALF) & (jnp.abs(y) < HALF)
                        & (jnp.abs(z) < HALF))
                ix = jnp.clip((x / STEP + 50.0).astype(jnp.int32), 0, NB - 1)
                iy = jnp.clip((y / STEP + 50.0).astype(jnp.int32), 0, NB - 1)
                iz = jnp.clip((z / STEP + 50.0).astype(jnp.int32), 0, NB - 1)
                flat = (ix * NB + iy) * NB + iz
                flat = jnp.where(cond, flat, DUMMY)
                idx_v[s] = flat

            pltpu.async_copy(vox_hbm.at[idx_v], rows_v, sem).wait()

            @pl.loop(0, C // L)
            def _act(g):
                s = pl.ds(g * L, L)
                pt = iota + g * L
                for c in range(3):
                    v = plsc.load_gather(rows_v, [pt, csplat[c]])
                    out_v[c, s] = 1.0 / (1.0 + jnp.exp(-v))
                d = plsc.load_gather(rows_v, [pt, csplat[3]])
                out_v[3, s] = jnp.maximum(d, 0.0)

            for c in range(4):
                pltpu.sync_copy(out_v.at[c], out_hbm.at[c, pl.ds(start, C)])

    return sc_kernel


_N_POINTS = 4194304
_SC_KERNEL = _make_sc_kernel(_N_POINTS)


def kernel(xyz, voxels):
    vox = voxels.reshape(-1, 4)
    vox = jnp.pad(vox, ((0, PAD_ROWS), (0, ROW - 4)))
    out = _SC_KERNEL(xyz.T, vox)
    return out.T
